# baseline (speedup 1.0000x reference)
import functools

import jax
import jax.numpy as jnp
from jax import lax
from jax.experimental import pallas as pl
from jax.experimental.pallas import tpu as pltpu
from jax.experimental.pallas import tpu_sc as plsc

N = 10000
D = 128
E = 320000

NUM_CORES = 2
NUM_SUBCORES = 16
NUM_WORKERS = NUM_CORES * NUM_SUBCORES

CHUNK = 40
NP = 10368
PAD_ROWS = NP - N

N_CHUNKS = 256
EDGES_PER_TILE = N_CHUNKS * CHUNK
EP = EDGES_PER_TILE * NUM_WORKERS
ROWS_PER_TILE_NP = NP // NUM_SUBCORES


NBUF = 8
SLAB = 16
N_SLABS = N_CHUNKS // SLAB
SUPERS_PER_SLAB = (SLAB - NBUF) // NBUF


def _make_sc_aggregate():
    mesh = plsc.VectorSubcoreMesh(core_axis_name="c", subcore_axis_name="s")

    @functools.partial(
        pl.kernel,
        out_type=jax.ShapeDtypeStruct((NUM_CORES, NP, D), jnp.float32),
        mesh=mesh,
        scratch_types=[
            pltpu.VMEM((SLAB, CHUNK), jnp.int32),
            pltpu.VMEM((SLAB, CHUNK), jnp.int32),
            [pltpu.VMEM((CHUNK, D), jnp.float32) for _ in range(NBUF)],
            pltpu.VMEM_SHARED((NP, D), jnp.float32),
            [pltpu.SemaphoreType.DMA for _ in range(NBUF)],
            [pltpu.SemaphoreType.DMA for _ in range(NBUF)],
        ],
    )
    def agg(feat_hbm, src_hbm, dst_hbm, zeros_hbm, out_hbm,
            src_v, dst_v, rows, acc, gsems, ssems):
        cid = lax.axis_index("c")
        sid = lax.axis_index("s")
        wid = cid * NUM_SUBCORES + sid

        pltpu.sync_copy(
            zeros_hbm.at[pl.ds(sid * ROWS_PER_TILE_NP, ROWS_PER_TILE_NP)],
            acc.at[pl.ds(sid * ROWS_PER_TILE_NP, ROWS_PER_TILE_NP)],
        )
        plsc.subcore_barrier()

        def fire_gather(b, i):
            pltpu.async_copy(feat_hbm.at[src_v.at[i]], rows[b], gsems[b])

        def wait_gather(b, i):
            pltpu.make_async_copy(feat_hbm.at[src_v.at[i]], rows[b],
                                  gsems[b]).wait()

        def fire_scatter(b, i):
            pltpu.async_copy(rows[b], acc.at[dst_v.at[i]], ssems[b],
                             add=True)

        def wait_scatter(b, i):
            pltpu.make_async_copy(rows[b], acc.at[dst_v.at[i]],
                                  ssems[b]).wait()

        def slab_body(t, _):
            pltpu.sync_copy(src_hbm.at[wid, t], src_v)
            pltpu.sync_copy(dst_hbm.at[wid, t], dst_v)
            for b in range(NBUF):
                fire_gather(b, b)

            def super_body(s, _):
                i0 = s * NBUF
                for b in range(NBUF):
                    wait_gather(b, i0 + b)
                    fire_scatter(b, i0 + b)
                for b in range(NBUF):
                    wait_scatter(b, i0 + b)
                    fire_gather(b, i0 + NBUF + b)
                return 0

            lax.fori_loop(0, SUPERS_PER_SLAB, super_body, 0)
            i0 = SUPERS_PER_SLAB * NBUF
            for b in range(NBUF):
                wait_gather(b, i0 + b)
                fire_scatter(b, i0 + b)
            for b in range(NBUF):
                wait_scatter(b, i0 + b)
            return 0

        lax.fori_loop(0, N_SLABS, slab_body, 0)
        plsc.subcore_barrier()

        pltpu.sync_copy(
            acc.at[pl.ds(sid * ROWS_PER_TILE_NP, ROWS_PER_TILE_NP)],
            out_hbm.at[cid, pl.ds(sid * ROWS_PER_TILE_NP, ROWS_PER_TILE_NP)],
        )

    return agg


_sc_aggregate = _make_sc_aggregate()


def _make_tc_linear(apply_relu: bool):
    BLK = 1000

    def body(p0_ref, p1_ref, w_ref, b_ref, out_ref):
        s = p0_ref[0] + p1_ref[0]
        y = jnp.dot(s, w_ref[...], preferred_element_type=jnp.float32,
                    precision=lax.Precision.HIGHEST)
        y = y + b_ref[...]
        if apply_relu:
            y = jnp.maximum(y, 0.0)
        out_ref[...] = y

    return pl.pallas_call(
        body,
        grid=(N // BLK,),
        in_specs=[
            pl.BlockSpec((1, BLK, D), lambda i: (0, i, 0)),
            pl.BlockSpec((1, BLK, D), lambda i: (1, i, 0)),
            pl.BlockSpec((D, D), lambda i: (0, 0)),
            pl.BlockSpec((1, D), lambda i: (0, 0)),
        ],
        out_specs=pl.BlockSpec((BLK, D), lambda i: (i, 0)),
        out_shape=jax.ShapeDtypeStruct((N, D), jnp.float32),
    )


_tc_linear_relu = _make_tc_linear(True)
_tc_linear = _make_tc_linear(False)


def kernel(x, edge_index, W1, b1, W2, b2):
    src = edge_index[0]
    dst = edge_index[1]

    n_pad = EP - E
    pad_ids = lax.iota(jnp.int32, n_pad)
    srcp = jnp.concatenate([src, pad_ids % N]).reshape(
        NUM_WORKERS, N_SLABS, SLAB, CHUNK)
    dstp = jnp.concatenate([dst, N + (pad_ids % PAD_ROWS)]).reshape(
        NUM_WORKERS, N_SLABS, SLAB, CHUNK)

    zeros = jnp.zeros((NP, D), jnp.float32)
    b1r = b1.reshape(1, D)
    b2r = b2.reshape(1, D)

    parts1 = _sc_aggregate(x, srcp, dstp, zeros)
    f2 = _tc_linear_relu(parts1, parts1, W1, b1r)
    parts2 = _sc_aggregate(f2, srcp, dstp, zeros)
    f4 = _tc_linear(parts2, parts2, W2, b2r)
    return f4


# speedup vs baseline: 1.1902x; 1.1902x over previous
import functools

import jax
import jax.numpy as jnp
from jax import lax
from jax.experimental import pallas as pl
from jax.experimental.pallas import tpu as pltpu
from jax.experimental.pallas import tpu_sc as plsc

N = 10000
D = 128
E = 320000

NUM_CORES = 2
NUM_SUBCORES = 16
NUM_WORKERS = NUM_CORES * NUM_SUBCORES

CHUNK = 64
NP = 10368
PAD_ROWS = NP - N

N_CHUNKS = 160
EDGES_PER_TILE = N_CHUNKS * CHUNK
EP = EDGES_PER_TILE * NUM_WORKERS
ROWS_PER_TILE_NP = NP // NUM_SUBCORES


NBUF = 4
SLAB = 40
N_SLABS = N_CHUNKS // SLAB
SUPERS_PER_SLAB = (SLAB - NBUF) // NBUF


def _make_sc_aggregate():
    mesh = plsc.VectorSubcoreMesh(core_axis_name="c", subcore_axis_name="s")

    @functools.partial(
        pl.kernel,
        out_type=jax.ShapeDtypeStruct((NUM_CORES, NP, D), jnp.float32),
        mesh=mesh,
        scratch_types=[
            pltpu.VMEM((SLAB, CHUNK), jnp.int32),
            pltpu.VMEM((SLAB, CHUNK), jnp.int32),
            [pltpu.VMEM((CHUNK, D), jnp.float32) for _ in range(NBUF)],
            pltpu.VMEM_SHARED((NP, D), jnp.float32),
            [pltpu.SemaphoreType.DMA for _ in range(NBUF)],
            [pltpu.SemaphoreType.DMA for _ in range(NBUF)],
        ],
    )
    def agg(feat_hbm, src_hbm, dst_hbm, zeros_hbm, out_hbm,
            src_v, dst_v, rows, acc, gsems, ssems):
        cid = lax.axis_index("c")
        sid = lax.axis_index("s")
        wid = cid * NUM_SUBCORES + sid

        def fire_gather(b, i):
            pltpu.async_copy(feat_hbm.at[src_v.at[i]], rows[b], gsems[b])

        def wait_gather(b, i):
            pltpu.make_async_copy(feat_hbm.at[src_v.at[i]], rows[b],
                                  gsems[b]).wait()

        def fire_scatter(b, i):
            pltpu.async_copy(rows[b], acc.at[dst_v.at[i]], ssems[b],
                             add=True)

        def wait_scatter(b, i):
            pltpu.make_async_copy(rows[b], acc.at[dst_v.at[i]],
                                  ssems[b]).wait()

        def slab_body(t, _):
            pltpu.sync_copy(src_hbm.at[wid, t], src_v)
            pltpu.sync_copy(dst_hbm.at[wid, t], dst_v)
            for b in range(NBUF):
                fire_gather(b, b)

            @pl.when(t == 0)
            def _():
                pltpu.sync_copy(
                    zeros_hbm.at[pl.ds(sid * ROWS_PER_TILE_NP,
                                       ROWS_PER_TILE_NP)],
                    acc.at[pl.ds(sid * ROWS_PER_TILE_NP, ROWS_PER_TILE_NP)],
                )
                plsc.subcore_barrier()

            def super_body(s, _):
                i0 = s * NBUF
                for b in range(NBUF):
                    wait_gather(b, i0 + b)
                    fire_scatter(b, i0 + b)
                for b in range(NBUF):
                    wait_scatter(b, i0 + b)
                    fire_gather(b, i0 + NBUF + b)
                return 0

            lax.fori_loop(0, SUPERS_PER_SLAB, super_body, 0)
            i0 = SUPERS_PER_SLAB * NBUF
            for b in range(NBUF):
                wait_gather(b, i0 + b)
                fire_scatter(b, i0 + b)
            for b in range(NBUF):
                wait_scatter(b, i0 + b)
            return 0

        lax.fori_loop(0, N_SLABS, slab_body, 0)
        plsc.subcore_barrier()

        pltpu.sync_copy(
            acc.at[pl.ds(sid * ROWS_PER_TILE_NP, ROWS_PER_TILE_NP)],
            out_hbm.at[cid, pl.ds(sid * ROWS_PER_TILE_NP, ROWS_PER_TILE_NP)],
        )

    return agg


_sc_aggregate = _make_sc_aggregate()


def _make_tc_linear(apply_relu: bool):
    BLK = 1000

    def body(p0_ref, p1_ref, w_ref, b_ref, out_ref):
        s = p0_ref[0] + p1_ref[0]
        y = jnp.dot(s, w_ref[...], preferred_element_type=jnp.float32)
        y = y + b_ref[...]
        if apply_relu:
            y = jnp.maximum(y, 0.0)
        out_ref[...] = y

    return pl.pallas_call(
        body,
        grid=(N // BLK,),
        in_specs=[
            pl.BlockSpec((1, BLK, D), lambda i: (0, i, 0)),
            pl.BlockSpec((1, BLK, D), lambda i: (1, i, 0)),
            pl.BlockSpec((D, D), lambda i: (0, 0)),
            pl.BlockSpec((1, D), lambda i: (0, 0)),
        ],
        out_specs=pl.BlockSpec((BLK, D), lambda i: (i, 0)),
        out_shape=jax.ShapeDtypeStruct((N, D), jnp.float32),
    )


_tc_linear_relu = _make_tc_linear(True)
_tc_linear = _make_tc_linear(False)


def kernel(x, edge_index, W1, b1, W2, b2):
    src = edge_index[0]
    dst = edge_index[1]

    n_pad = EP - E
    pad_ids = lax.iota(jnp.int32, n_pad)
    srcp = jnp.concatenate([src, pad_ids % N]).reshape(
        NUM_WORKERS, N_SLABS, SLAB, CHUNK)
    dstp = jnp.concatenate([dst, N + (pad_ids % PAD_ROWS)]).reshape(
        NUM_WORKERS, N_SLABS, SLAB, CHUNK)

    zeros = jnp.zeros((NP, D), jnp.float32)
    b1r = b1.reshape(1, D)
    b2r = b2.reshape(1, D)

    parts1 = _sc_aggregate(x, srcp, dstp, zeros)
    f2 = _tc_linear_relu(parts1, parts1, W1, b1r)
    parts2 = _sc_aggregate(f2, srcp, dstp, zeros)
    f4 = _tc_linear(parts2, parts2, W2, b2r)
    return f4


# speedup vs baseline: 1.2441x; 1.0452x over previous
import functools

import jax
import jax.numpy as jnp
from jax import lax
from jax.experimental import pallas as pl
from jax.experimental.pallas import tpu as pltpu
from jax.experimental.pallas import tpu_sc as plsc

N = 10000
D = 128
E = 320000

NUM_CORES = 2
NUM_SUBCORES = 16
NUM_WORKERS = NUM_CORES * NUM_SUBCORES

CHUNK = 64
NP = 10368
PAD_ROWS = NP - N

N_CHUNKS = 160
EDGES_PER_TILE = N_CHUNKS * CHUNK
EP = EDGES_PER_TILE * NUM_WORKERS
ROWS_PER_TILE_NP = NP // NUM_SUBCORES


NBUF = 4
SLAB = 20
N_SLABS = N_CHUNKS // SLAB
FULL_SUPERS = SLAB // NBUF - 1


def _make_sc_aggregate():
    mesh = plsc.VectorSubcoreMesh(core_axis_name="c", subcore_axis_name="s")

    @functools.partial(
        pl.kernel,
        out_type=jax.ShapeDtypeStruct((NUM_CORES, NP, D), jnp.float32),
        mesh=mesh,
        scratch_types=[
            [pltpu.VMEM((SLAB, CHUNK), jnp.int32) for _ in range(2)],
            [pltpu.VMEM((SLAB, CHUNK), jnp.int32) for _ in range(2)],
            [pltpu.VMEM((CHUNK, D), jnp.float32) for _ in range(NBUF)],
            pltpu.VMEM_SHARED((NP, D), jnp.float32),
            [pltpu.SemaphoreType.DMA for _ in range(NBUF)],
            [pltpu.SemaphoreType.DMA for _ in range(NBUF)],
            [pltpu.SemaphoreType.DMA for _ in range(2)],
        ],
    )
    def agg(feat_hbm, src_hbm, dst_hbm, zeros_hbm, out_hbm,
            src_J, dst_J, rows, acc, gsems, ssems, isems):
        cid = lax.axis_index("c")
        sid = lax.axis_index("s")
        wid = cid * NUM_SUBCORES + sid

        def fire_gather(p, b, r):
            pltpu.async_copy(feat_hbm.at[src_J[p].at[r]], rows[b], gsems[b])

        def wait_gather(p, b, r):
            pltpu.make_async_copy(feat_hbm.at[src_J[p].at[r]], rows[b],
                                  gsems[b]).wait()

        def fire_scatter(p, b, r):
            pltpu.async_copy(rows[b], acc.at[dst_J[p].at[r]], ssems[b],
                             add=True)

        def wait_scatter(p, b, r):
            pltpu.make_async_copy(rows[b], acc.at[dst_J[p].at[r]],
                                  ssems[b]).wait()

        def fire_idx(t, p):
            pltpu.async_copy(src_hbm.at[wid, t], src_J[p], isems[p])
            pltpu.async_copy(dst_hbm.at[wid, t], dst_J[p], isems[p])

        def wait_idx(t, p):
            pltpu.make_async_copy(src_hbm.at[wid, t], src_J[p],
                                  isems[p]).wait()
            pltpu.make_async_copy(dst_hbm.at[wid, t], dst_J[p],
                                  isems[p]).wait()

        pltpu.sync_copy(src_hbm.at[wid, 0], src_J[0])
        pltpu.sync_copy(dst_hbm.at[wid, 0], dst_J[0])
        fire_idx(1, 1)
        for b in range(NBUF):
            fire_gather(0, b, b)

        pltpu.sync_copy(
            zeros_hbm.at[pl.ds(sid * ROWS_PER_TILE_NP, ROWS_PER_TILE_NP)],
            acc.at[pl.ds(sid * ROWS_PER_TILE_NP, ROWS_PER_TILE_NP)],
        )
        plsc.subcore_barrier()

        def slab_proc(t, p):
            def super_body(s, _):
                i0 = s * NBUF
                for b in range(NBUF):
                    wait_gather(p, b, i0 + b)
                    fire_scatter(p, b, i0 + b)
                for b in range(NBUF):
                    wait_scatter(p, b, i0 + b)
                    fire_gather(p, b, i0 + NBUF + b)
                return 0

            lax.fori_loop(0, FULL_SUPERS, super_body, 0)

            not_last = t + 1 < N_SLABS

            @pl.when(not_last)
            def _():
                wait_idx(t + 1, 1 - p)

            i0 = FULL_SUPERS * NBUF
            for b in range(NBUF):
                wait_gather(p, b, i0 + b)
                fire_scatter(p, b, i0 + b)
            for b in range(NBUF):
                wait_scatter(p, b, i0 + b)

                @pl.when(not_last)
                def _():
                    fire_gather(1 - p, b, b)

            @pl.when(t + 2 < N_SLABS)
            def _():
                fire_idx(t + 2, p)

        def double_slab(tt, _):
            slab_proc(2 * tt, 0)
            slab_proc(2 * tt + 1, 1)
            return 0

        lax.fori_loop(0, N_SLABS // 2, double_slab, 0)
        plsc.subcore_barrier()

        pltpu.sync_copy(
            acc.at[pl.ds(sid * ROWS_PER_TILE_NP, ROWS_PER_TILE_NP)],
            out_hbm.at[cid, pl.ds(sid * ROWS_PER_TILE_NP, ROWS_PER_TILE_NP)],
        )

    return agg


_sc_aggregate = _make_sc_aggregate()


def _make_tc_linear(apply_relu: bool):
    BLK = 1000

    def body(p0_ref, p1_ref, w_ref, b_ref, out_ref):
        s = p0_ref[0] + p1_ref[0]
        y = jnp.dot(s, w_ref[...], preferred_element_type=jnp.float32)
        y = y + b_ref[...]
        if apply_relu:
            y = jnp.maximum(y, 0.0)
        out_ref[...] = y

    return pl.pallas_call(
        body,
        grid=(N // BLK,),
        in_specs=[
            pl.BlockSpec((1, BLK, D), lambda i: (0, i, 0)),
            pl.BlockSpec((1, BLK, D), lambda i: (1, i, 0)),
            pl.BlockSpec((D, D), lambda i: (0, 0)),
            pl.BlockSpec((1, D), lambda i: (0, 0)),
        ],
        out_specs=pl.BlockSpec((BLK, D), lambda i: (i, 0)),
        out_shape=jax.ShapeDtypeStruct((N, D), jnp.float32),
    )


_tc_linear_relu = _make_tc_linear(True)
_tc_linear = _make_tc_linear(False)


def kernel(x, edge_index, W1, b1, W2, b2):
    src = edge_index[0]
    dst = edge_index[1]

    n_pad = EP - E
    pad_ids = lax.iota(jnp.int32, n_pad)
    srcp = jnp.concatenate([src, pad_ids % N]).reshape(
        NUM_WORKERS, N_SLABS, SLAB, CHUNK)
    dstp = jnp.concatenate([dst, N + (pad_ids % PAD_ROWS)]).reshape(
        NUM_WORKERS, N_SLABS, SLAB, CHUNK)

    zeros = jnp.zeros((NP, D), jnp.float32)
    b1r = b1.reshape(1, D)
    b2r = b2.reshape(1, D)

    parts1 = _sc_aggregate(x, srcp, dstp, zeros)
    f2 = _tc_linear_relu(parts1, parts1, W1, b1r)
    parts2 = _sc_aggregate(f2, srcp, dstp, zeros)
    f4 = _tc_linear(parts2, parts2, W2, b2r)
    return f4


# speedup vs baseline: 1.2747x; 1.0247x over previous
import functools

import jax
import jax.numpy as jnp
from jax import lax
from jax.experimental import pallas as pl
from jax.experimental.pallas import tpu as pltpu
from jax.experimental.pallas import tpu_sc as plsc

N = 10000
D = 128
E = 320000

NUM_CORES = 2
NUM_SUBCORES = 16
NUM_WORKERS = NUM_CORES * NUM_SUBCORES

CHUNK = 64
NP = 10112
PAD_ROWS = NP - N

N_CHUNKS = 160
EDGES_PER_TILE = N_CHUNKS * CHUNK
EP = EDGES_PER_TILE * NUM_WORKERS
ROWS_PER_TILE_NP = NP // NUM_SUBCORES


NBUF = 5
SLAB = 10
N_SLABS = N_CHUNKS // SLAB
FULL_SUPERS = SLAB // NBUF - 1


def _make_sc_aggregate():
    mesh = plsc.VectorSubcoreMesh(core_axis_name="c", subcore_axis_name="s")

    @functools.partial(
        pl.kernel,
        out_type=jax.ShapeDtypeStruct((NUM_CORES, NP, D), jnp.float32),
        mesh=mesh,
        scratch_types=[
            [pltpu.VMEM((SLAB, CHUNK), jnp.int32) for _ in range(2)],
            [pltpu.VMEM((SLAB, CHUNK), jnp.int32) for _ in range(2)],
            [pltpu.VMEM((CHUNK, D), jnp.float32) for _ in range(NBUF)],
            pltpu.VMEM_SHARED((NP, D), jnp.float32),
            [pltpu.SemaphoreType.DMA for _ in range(NBUF)],
            [pltpu.SemaphoreType.DMA for _ in range(NBUF)],
            [pltpu.SemaphoreType.DMA for _ in range(2)],
        ],
    )
    def agg(feat_hbm, src_hbm, dst_hbm, zeros_hbm, out_hbm,
            src_J, dst_J, rows, acc, gsems, ssems, isems):
        cid = lax.axis_index("c")
        sid = lax.axis_index("s")
        wid = cid * NUM_SUBCORES + sid

        def fire_gather(p, b, r):
            pltpu.async_copy(feat_hbm.at[src_J[p].at[r]], rows[b], gsems[b])

        def wait_gather(p, b, r):
            pltpu.make_async_copy(feat_hbm.at[src_J[p].at[r]], rows[b],
                                  gsems[b]).wait()

        def fire_scatter(p, b, r):
            pltpu.async_copy(rows[b], acc.at[dst_J[p].at[r]], ssems[b],
                             add=True)

        def wait_scatter(p, b, r):
            pltpu.make_async_copy(rows[b], acc.at[dst_J[p].at[r]],
                                  ssems[b]).wait()

        def fire_idx(t, p):
            pltpu.async_copy(src_hbm.at[wid, t], src_J[p], isems[p])
            pltpu.async_copy(dst_hbm.at[wid, t], dst_J[p], isems[p])

        def wait_idx(t, p):
            pltpu.make_async_copy(src_hbm.at[wid, t], src_J[p],
                                  isems[p]).wait()
            pltpu.make_async_copy(dst_hbm.at[wid, t], dst_J[p],
                                  isems[p]).wait()

        pltpu.sync_copy(src_hbm.at[wid, 0], src_J[0])
        pltpu.sync_copy(dst_hbm.at[wid, 0], dst_J[0])
        fire_idx(1, 1)
        for b in range(NBUF):
            fire_gather(0, b, b)

        pltpu.sync_copy(
            zeros_hbm.at[pl.ds(sid * ROWS_PER_TILE_NP, ROWS_PER_TILE_NP)],
            acc.at[pl.ds(sid * ROWS_PER_TILE_NP, ROWS_PER_TILE_NP)],
        )
        plsc.subcore_barrier()

        def slab_proc(t, p):
            def super_body(s, _):
                i0 = s * NBUF
                for b in range(NBUF):
                    wait_gather(p, b, i0 + b)
                    fire_scatter(p, b, i0 + b)
                for b in range(NBUF):
                    wait_scatter(p, b, i0 + b)
                    fire_gather(p, b, i0 + NBUF + b)
                return 0

            lax.fori_loop(0, FULL_SUPERS, super_body, 0)

            not_last = t + 1 < N_SLABS

            @pl.when(not_last)
            def _():
                wait_idx(t + 1, 1 - p)

            i0 = FULL_SUPERS * NBUF
            for b in range(NBUF):
                wait_gather(p, b, i0 + b)
                fire_scatter(p, b, i0 + b)
            for b in range(NBUF):
                wait_scatter(p, b, i0 + b)

                @pl.when(not_last)
                def _():
                    fire_gather(1 - p, b, b)

            @pl.when(t + 2 < N_SLABS)
            def _():
                fire_idx(t + 2, p)

        def double_slab(tt, _):
            slab_proc(2 * tt, 0)
            slab_proc(2 * tt + 1, 1)
            return 0

        lax.fori_loop(0, N_SLABS // 2, double_slab, 0)
        plsc.subcore_barrier()

        pltpu.sync_copy(
            acc.at[pl.ds(sid * ROWS_PER_TILE_NP, ROWS_PER_TILE_NP)],
            out_hbm.at[cid, pl.ds(sid * ROWS_PER_TILE_NP, ROWS_PER_TILE_NP)],
        )

    return agg


_sc_aggregate = _make_sc_aggregate()


def _make_tc_linear(apply_relu: bool):
    BLK = 1000

    def body(p0_ref, p1_ref, w_ref, b_ref, out_ref):
        s = p0_ref[0] + p1_ref[0]
        y = jnp.dot(s, w_ref[...], preferred_element_type=jnp.float32)
        y = y + b_ref[...]
        if apply_relu:
            y = jnp.maximum(y, 0.0)
        out_ref[...] = y

    return pl.pallas_call(
        body,
        grid=(N // BLK,),
        in_specs=[
            pl.BlockSpec((1, BLK, D), lambda i: (0, i, 0)),
            pl.BlockSpec((1, BLK, D), lambda i: (1, i, 0)),
            pl.BlockSpec((D, D), lambda i: (0, 0)),
            pl.BlockSpec((1, D), lambda i: (0, 0)),
        ],
        out_specs=pl.BlockSpec((BLK, D), lambda i: (i, 0)),
        out_shape=jax.ShapeDtypeStruct((N, D), jnp.float32),
    )


_tc_linear_relu = _make_tc_linear(True)
_tc_linear = _make_tc_linear(False)


def kernel(x, edge_index, W1, b1, W2, b2):
    src = edge_index[0]
    dst = edge_index[1]

    n_pad = EP - E
    pad_ids = lax.iota(jnp.int32, n_pad)
    srcp = jnp.concatenate([src, pad_ids % N]).reshape(
        NUM_WORKERS, N_SLABS, SLAB, CHUNK)
    dstp = jnp.concatenate([dst, N + (pad_ids % PAD_ROWS)]).reshape(
        NUM_WORKERS, N_SLABS, SLAB, CHUNK)

    zeros = jnp.zeros((NP, D), jnp.float32)
    b1r = b1.reshape(1, D)
    b2r = b2.reshape(1, D)

    parts1 = _sc_aggregate(x, srcp, dstp, zeros)
    f2 = _tc_linear_relu(parts1, parts1, W1, b1r)
    parts2 = _sc_aggregate(f2, srcp, dstp, zeros)
    f4 = _tc_linear(parts2, parts2, W2, b2r)
    return f4
